# Initial kernel scaffold; baseline (speedup 1.0000x reference)
import jax
import jax.numpy as jnp
from jax.experimental import pallas as pl


def kernel(x, points):
    raise NotImplementedError("write your pallas kernel here")


# speedup vs baseline: 2.8559x; 2.8559x over previous
import functools

import jax
import jax.numpy as jnp
from jax import lax
from jax.experimental import pallas as pl
from jax.experimental.pallas import tpu as pltpu
from jax.experimental.pallas import tpu_sc as plsc

B = 8
H = 224
W = 224
HW = H * W
NPTS = 2048
NTILES = 32
TPB = NTILES // B
SEG = HW // TPB
L = 16


def _sc_clickmap(points):
    mesh = plsc.VectorSubcoreMesh(core_axis_name="c", subcore_axis_name="s")

    @functools.partial(
        pl.kernel,
        mesh=mesh,
        out_type=jax.ShapeDtypeStruct((B, HW), jnp.float32),
        scratch_types=[
            pltpu.VMEM((NPTS * 2,), jnp.int32),
            pltpu.VMEM((SEG,), jnp.float32),
        ],
        compiler_params=pltpu.CompilerParams(needs_layout_passes=False),
    )
    def clickmap_kernel(points_hbm, out_hbm, pts_v, seg_v):
        cid = lax.axis_index("c")
        sid = lax.axis_index("s")
        wid = sid * 2 + cid
        batch = wid // TPB
        lo = (wid % TPB) * SEG

        pltpu.sync_copy(points_hbm.at[batch], pts_v)

        zeros_f = jnp.zeros((L,), jnp.float32)

        def zero_body(i, carry):
            seg_v[pl.ds(i * L, L)] = zeros_f
            return carry

        lax.fori_loop(0, SEG // L, zero_body, 0)

        iot = lax.iota(jnp.int32, L)
        ones_f = jnp.ones((L,), jnp.float32)

        def scatter_body(j, carry):
            ivec = (j * L + iot) * 2
            r = plsc.load_gather(pts_v, [ivec])
            c = plsc.load_gather(pts_v, [ivec + 1])
            local = r * W + c - lo
            mask = (local >= 0) & (local < SEG)
            safe = jnp.where(mask, local, 0)
            plsc.store_scatter(seg_v, [safe], ones_f, mask=mask)
            return carry

        lax.fori_loop(0, NPTS // L, scatter_body, 0)

        pltpu.sync_copy(seg_v, out_hbm.at[batch, pl.ds(lo, SEG)])

    return clickmap_kernel(points.reshape(B, NPTS * 2))


def kernel(x, points):
    del x
    return _sc_clickmap(points).reshape(B, 1, H, W)


# speedup vs baseline: 3.2879x; 1.1513x over previous
import functools

import jax
import jax.numpy as jnp
from jax import lax
from jax.experimental import pallas as pl
from jax.experimental.pallas import tpu as pltpu
from jax.experimental.pallas import tpu_sc as plsc

B = 8
H = 224
W = 224
HW = H * W
NPTS = 2048
NTILES = 32
TPB = NTILES // B
SEG = HW // TPB
L = 16


def _sc_clickmap(points):
    mesh = plsc.VectorSubcoreMesh(core_axis_name="c", subcore_axis_name="s")

    @functools.partial(
        pl.kernel,
        mesh=mesh,
        out_type=jax.ShapeDtypeStruct((B, HW), jnp.float32),
        scratch_types=[
            pltpu.VMEM((NPTS * 2,), jnp.int32),
            pltpu.VMEM((SEG,), jnp.float32),
            pltpu.SemaphoreType.DMA,
        ],
        compiler_params=pltpu.CompilerParams(needs_layout_passes=False),
    )
    def clickmap_kernel(points_hbm, out_hbm, pts_v, seg_v, sem):
        cid = lax.axis_index("c")
        sid = lax.axis_index("s")
        wid = sid * 2 + cid
        batch = wid // TPB
        lo = (wid % TPB) * SEG

        cp = pltpu.async_copy(points_hbm.at[batch], pts_v, sem)

        zeros_f = jnp.zeros((L,), jnp.float32)

        @plsc.parallel_loop(0, SEG // L, unroll=8)
        def _zero(i):
            seg_v[pl.ds(i * L, L)] = zeros_f

        cp.wait()

        iot = lax.iota(jnp.int32, L)
        ones_f = jnp.ones((L,), jnp.float32)

        @plsc.parallel_loop(0, NPTS // L, unroll=4)
        def _scatter(j):
            ivec = (j * L + iot) * 2
            r = plsc.load_gather(pts_v, [ivec])
            c = plsc.load_gather(pts_v, [ivec + 1])
            local = r * W + c - lo
            mask = (local >= 0) & (local < SEG)
            safe = jnp.where(mask, local, 0)
            plsc.store_scatter(seg_v, [safe], ones_f, mask=mask)

        pltpu.sync_copy(seg_v, out_hbm.at[batch, pl.ds(lo, SEG)])

    return clickmap_kernel(points.reshape(B, NPTS * 2))


def kernel(x, points):
    del x
    return _sc_clickmap(points).reshape(B, 1, H, W)
